# baseline (speedup 1.0000x reference)
import jax
import jax.numpy as jnp
from jax import lax
from jax.experimental import pallas as pl
from jax.experimental.pallas import tpu as pltpu
from jax.experimental.pallas import tpu_sc as plsc

F = 16
P = 196
N = F * P
LANES = 16
CHUNKS = N // LANES
K = 8
NUM_DMAS = 12


def _body(table_hbm, out_hbm, table_v, rows_v, mixed_v, sem, msem):
    c = lax.axis_index("c")
    s = lax.axis_index("s")
    start = 392 * (s // 2) + 96 * (2 * (s % 2) + c) + 8 * (s % 2)

    pltpu.sync_copy(table_hbm, table_v)

    lane = lax.iota(jnp.int32, LANES)
    hi = s + (F - 1)

    @plsc.parallel_loop(0, CHUNKS, unroll=4)
    def _build(i):
        col = lane + i * LANES
        vals = plsc.load_gather(table_v, [hi - col // P])
        base = i * LANES
        for k in range(K):
            rows_v[k, pl.ds(base, LANES)] = vals

    copies = [
        pltpu.async_copy(rows_v, out_hbm.at[pl.ds(start + j * K, K)], sem)
        for j in range(NUM_DMAS)
    ]

    @pl.when(s < 4)
    def _mixed():
        m = 2 * s + c
        hia = 2 * m + (F - 1)

        @plsc.parallel_loop(0, CHUNKS, unroll=4)
        def _mbuild(i):
            off = (lane + i * LANES) // P
            va = plsc.load_gather(table_v, [hia - off])
            vb = plsc.load_gather(table_v, [hia + 1 - off])
            base = i * LANES
            for k in range(K // 2):
                mixed_v[k, pl.ds(base, LANES)] = va
            for k in range(K // 2, K):
                mixed_v[k, pl.ds(base, LANES)] = vb

        pltpu.async_copy(
            mixed_v, out_hbm.at[pl.ds(392 * m + 192, K)], msem
        ).wait()

    for cp in copies:
        cp.wait()


@jax.jit
def _fill(table32):
    run = pl.kernel(
        _body,
        out_type=jax.ShapeDtypeStruct((N, N), jnp.float32),
        mesh=plsc.VectorSubcoreMesh(core_axis_name="c", subcore_axis_name="s"),
        compiler_params=pltpu.CompilerParams(needs_layout_passes=False),
        scratch_types=[
            pltpu.VMEM((2 * F, ), jnp.float32),
            pltpu.VMEM((K, N), jnp.float32),
            pltpu.VMEM((K, N), jnp.float32),
            pltpu.SemaphoreType.DMA,
            pltpu.SemaphoreType.DMA,
        ],
    )
    return run(table32)


def kernel(temporal_embedding):
    table32 = jnp.pad(temporal_embedding.reshape(-1), (0, 1))
    return _fill(table32)


# speedup vs baseline: 1.3405x; 1.3405x over previous
import jax
import jax.numpy as jnp
from jax import lax
from jax.experimental import pallas as pl
from jax.experimental.pallas import tpu as pltpu
from jax.experimental.pallas import tpu_sc as plsc

F = 16
P = 196
N = F * P
LANES = 16
CHUNKS = N // LANES
K = 8
NUM_DMAS = 12


def _body(table_hbm, out_hbm, table_v, rows_v, mixed_v, sem, msem):
    c = lax.axis_index("c")
    s = lax.axis_index("s")
    start = 392 * (s // 2) + 96 * (2 * (s % 2) + c) + 8 * (s % 2)

    pltpu.sync_copy(table_hbm, table_v)

    lane = lax.iota(jnp.int32, LANES)
    zero = lane * 0
    hi = s + (F - 1)
    offs = [16 * t for t in range(12)] + [P - LANES]

    @plsc.parallel_loop(0, F)
    def _build(jf):
        vals = plsc.load_gather(table_v, [zero + (hi - jf)])
        base = jf * P
        for k in range(K):
            for t in offs:
                rows_v[k, pl.ds(base + t, LANES)] = vals

    copies = [
        pltpu.async_copy(rows_v, out_hbm.at[pl.ds(start + j * K, K)], sem)
        for j in range(NUM_DMAS)
    ]

    @pl.when(s < 4)
    def _mixed():
        m = 2 * s + c
        hia = 2 * m + (F - 1)

        @plsc.parallel_loop(0, F)
        def _mbuild(jf):
            va = plsc.load_gather(table_v, [zero + (hia - jf)])
            vb = plsc.load_gather(table_v, [zero + (hia + 1 - jf)])
            base = jf * P
            for k in range(K // 2):
                for t in offs:
                    mixed_v[k, pl.ds(base + t, LANES)] = va
            for k in range(K // 2, K):
                for t in offs:
                    mixed_v[k, pl.ds(base + t, LANES)] = vb

        pltpu.async_copy(
            mixed_v, out_hbm.at[pl.ds(392 * m + 192, K)], msem
        ).wait()

    for cp in copies:
        cp.wait()


@jax.jit
def _fill(table32):
    run = pl.kernel(
        _body,
        out_type=jax.ShapeDtypeStruct((N, N), jnp.float32),
        mesh=plsc.VectorSubcoreMesh(core_axis_name="c", subcore_axis_name="s"),
        compiler_params=pltpu.CompilerParams(needs_layout_passes=False),
        scratch_types=[
            pltpu.VMEM((2 * F, ), jnp.float32),
            pltpu.VMEM((K, N), jnp.float32),
            pltpu.VMEM((K, N), jnp.float32),
            pltpu.SemaphoreType.DMA,
            pltpu.SemaphoreType.DMA,
        ],
    )
    return run(table32)


def kernel(temporal_embedding):
    table32 = jnp.pad(temporal_embedding.reshape(-1), (0, 1))
    return _fill(table32)
